# baseline (speedup 1.0000x reference)
import functools

import jax
import jax.numpy as jnp
from jax.experimental import pallas as pl

N_ROWS_BLK = 2000
EMB = 128


def _sage_mm_body(h_ref, hn_ref, ws_ref, wn_ref, b_ref, o_ref):
    h = h_ref[...]
    hn = hn_ref[...]
    acc = jnp.dot(h, ws_ref[...], preferred_element_type=jnp.float32)
    acc += jnp.dot(hn, wn_ref[...], preferred_element_type=jnp.float32)
    o_ref[...] = acc + b_ref[...]


@functools.partial(jax.jit, static_argnames=("relu",))
def _sage_matmuls(h, h_neigh, Ws, Wn, b, relu=False):
    n = h.shape[0]
    grid = (n // N_ROWS_BLK,)
    out = pl.pallas_call(
        _sage_mm_body,
        grid=grid,
        in_specs=[
            pl.BlockSpec((N_ROWS_BLK, EMB), lambda i: (i, 0)),
            pl.BlockSpec((N_ROWS_BLK, EMB), lambda i: (i, 0)),
            pl.BlockSpec((EMB, EMB), lambda i: (0, 0)),
            pl.BlockSpec((EMB, EMB), lambda i: (0, 0)),
            pl.BlockSpec((1, EMB), lambda i: (0, 0)),
        ],
        out_specs=pl.BlockSpec((N_ROWS_BLK, EMB), lambda i: (i, 0)),
        out_shape=jax.ShapeDtypeStruct((n, EMB), jnp.float32),
    )(h, h_neigh, Ws, Wn, b.reshape(1, EMB))
    if relu:
        out = jax.nn.relu(out)
    return out


def kernel(node_idx, edge_index, edge_type, pre_embed, rel_weght,
           W_self0, W_neigh0, b0, W_self1, W_neigh1, b1):
    src = edge_index[0]
    dst = edge_index[1]
    n = node_idx.shape[0]
    e = src.shape[0]
    h = jnp.take(pre_embed, node_idx, axis=0)
    ew = jnp.take(rel_weght, edge_type, axis=0)
    deg = jax.ops.segment_sum(jnp.ones((e,), h.dtype), dst, num_segments=n)
    deg = jnp.maximum(deg, 1.0)

    def agg_layer(hh):
        m = jnp.take(hh, src, axis=0) * ew
        agg = jax.ops.segment_sum(m, dst, num_segments=n)
        return agg / deg[:, None]

    h1 = _sage_matmuls(h, agg_layer(h), W_self0, W_neigh0, b0, relu=True)
    h2 = _sage_matmuls(h1, agg_layer(h1), W_self1, W_neigh1, b1, relu=False)
    return h2


# speedup vs baseline: 2.2128x; 2.2128x over previous
import functools

import jax
import jax.numpy as jnp
from jax import lax
from jax.experimental import pallas as pl
from jax.experimental.pallas import tpu as pltpu
from jax.experimental.pallas import tpu_sc as plsc

N = 10000
E = 320000
D = 128
R = 32
NC, NS = 2, 16
NW = NC * NS
CH = 64

N_PAD = 10240
ROWS_PER_TILE = N_PAD // NS
NODE_PAD = 12288
E_PAD = -(-E // (NW * CH)) * NW * CH
DUMMY = N

_mesh = plsc.VectorSubcoreMesh(core_axis_name="c", subcore_axis_name="s")


def _zero_rows(buf, ncols):
    z = jnp.zeros((16,), jnp.float32)

    def row(i, _):
        for d in range(ncols // 16):
            buf[i, pl.ds(d * 16, 16)] = z
        return 0

    lax.fori_loop(0, buf.shape[0], row, 0)


@functools.partial(
    pl.kernel,
    out_type=jax.ShapeDtypeStruct((NODE_PAD, D), jnp.float32),
    mesh=_mesh,
    scratch_types=[
        pltpu.VMEM((CH,), jnp.int32),
        pltpu.VMEM((CH, D), jnp.float32),
        pltpu.SemaphoreType.DMA,
    ],
)
def _gather_nodes(table_hbm, nidx_hbm, out_hbm, idx_v, rows_v, sem):
    wid = lax.axis_index("c") * NS + lax.axis_index("s")
    for j in range(NODE_PAD // (NW * CH)):
        r = wid * (NODE_PAD // (NW * CH)) + j
        pltpu.sync_copy(nidx_hbm.at[r], idx_v)
        pltpu.async_copy(table_hbm.at[idx_v], rows_v, sem).wait()
        pltpu.sync_copy(rows_v, out_hbm.at[pl.ds(r * CH, CH)])


@functools.partial(
    pl.kernel,
    out_type=jax.ShapeDtypeStruct((NC, N_PAD, D), jnp.float32),
    mesh=_mesh,
    scratch_types=[
        pltpu.VMEM((CH,), jnp.int32),
        pltpu.VMEM((CH, D), jnp.float32),
        pltpu.VMEM_SHARED((N_PAD, D), jnp.float32),
    ],
)
def _deg_count(dst_hbm, out_hbm, didx, ones, deg_s):
    c = lax.axis_index("c")
    s = lax.axis_index("s")
    wid = c * NS + s
    base = s * ROWS_PER_TILE

    _zero_rows(ones, D)
    for k in range(ROWS_PER_TILE // CH):
        pltpu.sync_copy(ones, deg_s.at[pl.ds(base + k * CH, CH)])
    o = jnp.full((16,), 1.0, jnp.float32)

    def onesrow(i, _):
        ones[i, pl.ds(0, 16)] = o
        return 0

    lax.fori_loop(0, CH, onesrow, 0)
    plsc.subcore_barrier()

    nchunks = E_PAD // (NW * CH)

    def chunk(t, _):
        r = wid * nchunks + t
        pltpu.sync_copy(dst_hbm.at[r], didx)
        pltpu.sync_copy(ones, deg_s.at[didx], add=True)
        return 0

    lax.fori_loop(0, nchunks, chunk, 0)
    plsc.subcore_barrier()

    for k in range(ROWS_PER_TILE // CH):
        sl = pl.ds(base + k * CH, CH)
        pltpu.sync_copy(deg_s.at[sl], out_hbm.at[c, sl])


@functools.partial(
    pl.kernel,
    out_type=jax.ShapeDtypeStruct((NC, N_PAD, D), jnp.float32),
    mesh=_mesh,
    scratch_types=[
        pltpu.VMEM((CH,), jnp.int32),
        pltpu.VMEM((CH,), jnp.int32),
        pltpu.VMEM((CH,), jnp.int32),
        pltpu.VMEM((CH, D), jnp.float32),
        pltpu.VMEM((CH, D), jnp.float32),
        pltpu.VMEM_SHARED((N_PAD, D), jnp.float32),
        pltpu.SemaphoreType.DMA,
        pltpu.SemaphoreType.DMA,
    ],
)
def _edge_agg(h_hbm, rel_hbm, src_hbm, dst_hbm, typ_hbm, agg_out,
              sidx, didx, tidx, hrows, wrows, agg_s, sem1, sem2):
    c = lax.axis_index("c")
    s = lax.axis_index("s")
    wid = c * NS + s
    base = s * ROWS_PER_TILE

    _zero_rows(wrows, D)
    for k in range(ROWS_PER_TILE // CH):
        pltpu.sync_copy(wrows, agg_s.at[pl.ds(base + k * CH, CH)])
    plsc.subcore_barrier()

    nchunks = E_PAD // (NW * CH)

    def chunk(t, _):
        r = wid * nchunks + t
        pltpu.sync_copy(src_hbm.at[r], sidx)
        pltpu.sync_copy(dst_hbm.at[r], didx)
        pltpu.sync_copy(typ_hbm.at[r], tidx)
        pltpu.async_copy(h_hbm.at[sidx], hrows, sem1).wait()
        pltpu.async_copy(rel_hbm.at[tidx], wrows, sem2).wait()

        def mulrow(i, _):
            for d in range(D // 16):
                sl = pl.ds(d * 16, 16)
                hrows[i, sl] = hrows[i, sl] * wrows[i, sl]
            return 0

        lax.fori_loop(0, CH, mulrow, 0)
        pltpu.sync_copy(hrows, agg_s.at[didx], add=True)
        return 0

    lax.fori_loop(0, nchunks, chunk, 0)
    plsc.subcore_barrier()

    for k in range(ROWS_PER_TILE // CH):
        sl = pl.ds(base + k * CH, CH)
        pltpu.sync_copy(agg_s.at[sl], agg_out.at[c, sl])


N_BLK = 2000


def _make_mm_body(relu):
    def body(agg_ref, deg_ref, h_ref, ws_ref, wn_ref, b_ref, o_ref):
        agg = agg_ref[0] + agg_ref[1]
        d = deg_ref[0, :, :1] + deg_ref[1, :, :1]
        hn = agg / jnp.maximum(d, 1.0)
        acc = jnp.dot(h_ref[...], ws_ref[...], preferred_element_type=jnp.float32)
        acc += jnp.dot(hn, wn_ref[...], preferred_element_type=jnp.float32)
        acc += b_ref[...]
        o_ref[...] = jnp.maximum(acc, 0.0) if relu else acc

    return body


def _sage_mm(aggp, degp, h, Ws, Wn, b, relu):
    return pl.pallas_call(
        _make_mm_body(relu),
        grid=(N // N_BLK,),
        in_specs=[
            pl.BlockSpec((NC, N_BLK, D), lambda i: (0, i, 0)),
            pl.BlockSpec((NC, N_BLK, 8), lambda i: (0, i, 0)),
            pl.BlockSpec((N_BLK, D), lambda i: (i, 0)),
            pl.BlockSpec((D, D), lambda i: (0, 0)),
            pl.BlockSpec((D, D), lambda i: (0, 0)),
            pl.BlockSpec((1, D), lambda i: (0, 0)),
        ],
        out_specs=pl.BlockSpec((N_BLK, D), lambda i: (i, 0)),
        out_shape=jax.ShapeDtypeStruct((N, D), jnp.float32),
    )(aggp, degp, h, Ws, Wn, b.reshape(1, D))


@jax.jit
def _run(node_idx, src, dst, etype, pre_embed, rel_weght,
         W_self0, W_neigh0, b0, W_self1, W_neigh1, b1):
    i32 = jnp.int32
    nidx = jnp.concatenate(
        [node_idx.astype(i32), jnp.zeros((NODE_PAD - N,), i32)]
    ).reshape(NODE_PAD // CH, CH)
    src_p = jnp.concatenate(
        [src.astype(i32), jnp.zeros((E_PAD - E,), i32)]
    ).reshape(E_PAD // CH, CH)
    dst_p = jnp.concatenate(
        [dst.astype(i32), jnp.full((E_PAD - E,), DUMMY, i32)]
    ).reshape(E_PAD // CH, CH)
    typ_p = jnp.concatenate(
        [etype.astype(i32), jnp.zeros((E_PAD - E,), i32)]
    ).reshape(E_PAD // CH, CH)

    h0p = _gather_nodes(pre_embed, nidx)
    h0 = h0p[:N]

    degp = _deg_count(dst_p)[:, :, :8]
    agg0 = _edge_agg(h0, rel_weght, src_p, dst_p, typ_p)
    h1 = _sage_mm(agg0, degp, h0, W_self0, W_neigh0, b0, True)

    agg1 = _edge_agg(h1, rel_weght, src_p, dst_p, typ_p)
    h2 = _sage_mm(agg1, degp, h1, W_self1, W_neigh1, b1, False)
    return h2


def kernel(node_idx, edge_index, edge_type, pre_embed, rel_weght,
           W_self0, W_neigh0, b0, W_self1, W_neigh1, b1):
    return _run(node_idx, edge_index[0], edge_index[1], edge_type,
                pre_embed, rel_weght,
                W_self0, W_neigh0, b0, W_self1, W_neigh1, b1)


# speedup vs baseline: 5.2504x; 2.3728x over previous
import functools

import jax
import jax.numpy as jnp
from jax import lax
from jax.experimental import pallas as pl
from jax.experimental.pallas import tpu as pltpu
from jax.experimental.pallas import tpu_sc as plsc

N = 10000
E = 320000
D = 128
R = 32
NC, NS = 2, 16
NW = NC * NS
CH = 80
CHG = 128

N_PAD = 10240
ROWS_PER_TILE = N_PAD // NS
NODE_PAD = NW * 3 * CHG
E_PAD = NW * 126 * CH
NCHUNK = E_PAD // (NW * CH)
DUMMY = N

_mesh = plsc.VectorSubcoreMesh(core_axis_name="c", subcore_axis_name="s")


def _zero_rows(buf, ncols):
    z = jnp.zeros((16,), jnp.float32)

    def row(i, _):
        for d in range(ncols // 16):
            buf[i, pl.ds(d * 16, 16)] = z
        return 0

    lax.fori_loop(0, buf.shape[0], row, 0)


@functools.partial(
    pl.kernel,
    out_type=jax.ShapeDtypeStruct((NODE_PAD, D), jnp.float32),
    mesh=_mesh,
    scratch_types=[
        pltpu.VMEM((CHG,), jnp.int32),
        pltpu.VMEM((CHG,), jnp.int32),
        pltpu.VMEM((CHG, D), jnp.float32),
        pltpu.VMEM((CHG, D), jnp.float32),
        pltpu.SemaphoreType.DMA,
        pltpu.SemaphoreType.DMA,
    ],
)
def _gather_nodes(table_hbm, nidx_hbm, out_hbm, i0, i1, r0, r1, s0, s1):
    wid = lax.axis_index("c") * NS + lax.axis_index("s")
    idx = (i0, i1)
    rows = (r0, r1)
    sems = (s0, s1)
    nch = NODE_PAD // (NW * CHG)
    base = wid * nch
    pltpu.sync_copy(nidx_hbm.at[base], i0)
    pltpu.async_copy(table_hbm.at[i0], r0, s0)
    for j in range(nch):
        b = j % 2
        if j + 1 < nch:
            b2 = (j + 1) % 2
            pltpu.sync_copy(nidx_hbm.at[base + j + 1], idx[b2])
            pltpu.async_copy(table_hbm.at[idx[b2]], rows[b2], sems[b2])
        pltpu.make_async_copy(table_hbm.at[idx[b]], rows[b], sems[b]).wait()
        pltpu.sync_copy(rows[b], out_hbm.at[pl.ds((base + j) * CHG, CHG)])


@functools.partial(
    pl.kernel,
    out_type=jax.ShapeDtypeStruct((NC, N_PAD, D), jnp.float32),
    mesh=_mesh,
    scratch_types=[
        pltpu.VMEM((CH,), jnp.int32),
        pltpu.VMEM((CH,), jnp.int32),
        pltpu.VMEM((CH, D), jnp.float32),
        pltpu.VMEM_SHARED((N_PAD, D), jnp.float32),
        pltpu.SemaphoreType.DMA,
        pltpu.SemaphoreType.DMA,
    ],
)
def _deg_count(idx_hbm, out_hbm, d0, d1, ones, deg_s, s0, s1):
    c = lax.axis_index("c")
    s = lax.axis_index("s")
    wid = c * NS + s
    base = s * ROWS_PER_TILE

    _zero_rows(ones, D)
    for k in range(ROWS_PER_TILE // CH):
        pltpu.sync_copy(ones, deg_s.at[pl.ds(base + k * CH, CH)])
    o = jnp.full((16,), 1.0, jnp.float32)

    def onesrow(i, _):
        ones[i, pl.ds(0, 16)] = o
        return 0

    lax.fori_loop(0, CH, onesrow, 0)
    plsc.subcore_barrier()

    didx = (d0, d1)
    sems = (s0, s1)
    row0 = wid * NCHUNK

    def pair(p, _):
        for b in range(2):
            t = 2 * p + b

            @pl.when(p >= 1)
            def _():
                pltpu.make_async_copy(ones, deg_s.at[didx[b]], sems[b]).wait()

            pltpu.sync_copy(idx_hbm.at[row0 + t, 1], didx[b])
            pltpu.async_copy(ones, deg_s.at[didx[b]], sems[b], add=True)
        return 0

    lax.fori_loop(0, NCHUNK // 2, pair, 0)
    for b in range(2):
        pltpu.make_async_copy(ones, deg_s.at[didx[b]], sems[b]).wait()
    plsc.subcore_barrier()

    for k in range(ROWS_PER_TILE // CH):
        sl = pl.ds(base + k * CH, CH)
        pltpu.sync_copy(deg_s.at[sl], out_hbm.at[c, sl])


@functools.partial(
    pl.kernel,
    out_type=jax.ShapeDtypeStruct((NC, N_PAD, D), jnp.float32),
    mesh=_mesh,
    scratch_types=[
        pltpu.VMEM((3, CH), jnp.int32),
        pltpu.VMEM((3, CH), jnp.int32),
        pltpu.VMEM((CH, D), jnp.float32),
        pltpu.VMEM((CH, D), jnp.float32),
        pltpu.VMEM((CH, D), jnp.float32),
        pltpu.VMEM((CH, D), jnp.float32),
        pltpu.VMEM_SHARED((R, D), jnp.float32),
        pltpu.VMEM_SHARED((N_PAD, D), jnp.float32),
        pltpu.SemaphoreType.DMA,
        pltpu.SemaphoreType.DMA,
        pltpu.SemaphoreType.DMA,
        pltpu.SemaphoreType.DMA,
        pltpu.SemaphoreType.DMA,
        pltpu.SemaphoreType.DMA,
    ],
)
def _edge_agg(h_hbm, rel_hbm, idx_hbm, agg_out,
              ib0, ib1, hb0, hb1, wb0, wb1, rel_s, agg_s,
              hs0, hs1, ws0, ws1, ss0, ss1):
    c = lax.axis_index("c")
    s = lax.axis_index("s")
    wid = c * NS + s
    base = s * ROWS_PER_TILE

    _zero_rows(hb0, D)
    for k in range(ROWS_PER_TILE // CH):
        pltpu.sync_copy(hb0, agg_s.at[pl.ds(base + k * CH, CH)])

    @pl.when(s == 0)
    def _():
        pltpu.sync_copy(rel_hbm, rel_s)

    plsc.subcore_barrier()

    ib = (ib0, ib1)
    hb = (hb0, hb1)
    wb = (wb0, wb1)
    hs = (hs0, hs1)
    ws = (ws0, ws1)
    ss = (ss0, ss1)
    row0 = wid * NCHUNK

    pltpu.sync_copy(idx_hbm.at[row0], ib0)
    pltpu.async_copy(h_hbm.at[ib0.at[0]], hb0, hs0)
    pltpu.async_copy(rel_s.at[ib0.at[2]], wb0, ws0)

    def pair(p, _):
        for b in range(2):
            t = 2 * p + b
            b2 = 1 - b
            pltpu.make_async_copy(h_hbm.at[ib[b].at[0]], hb[b], hs[b]).wait()
            pltpu.make_async_copy(rel_s.at[ib[b].at[2]], wb[b], ws[b]).wait()

            @pl.when(t >= 1)
            def _():
                pltpu.make_async_copy(hb[b2], agg_s.at[ib[b2].at[1]], ss[b2]).wait()

            @pl.when(t + 1 < NCHUNK)
            def _():
                pltpu.sync_copy(idx_hbm.at[row0 + t + 1], ib[b2])
                pltpu.async_copy(h_hbm.at[ib[b2].at[0]], hb[b2], hs[b2])
                pltpu.async_copy(rel_s.at[ib[b2].at[2]], wb[b2], ws[b2])

            def mulrow(i, _):
                for d in range(D // 16):
                    sl = pl.ds(d * 16, 16)
                    hb[b][i, sl] = hb[b][i, sl] * wb[b][i, sl]
                return 0

            lax.fori_loop(0, CH, mulrow, 0)
            pltpu.async_copy(hb[b], agg_s.at[ib[b].at[1]], ss[b], add=True)
        return 0

    lax.fori_loop(0, NCHUNK // 2, pair, 0)
    pltpu.make_async_copy(hb1, agg_s.at[ib1.at[1]], ss1).wait()
    plsc.subcore_barrier()

    for k in range(ROWS_PER_TILE // CH):
        sl = pl.ds(base + k * CH, CH)
        pltpu.sync_copy(agg_s.at[sl], agg_out.at[c, sl])


N_BLK = 2000


def _make_mm_body(relu):
    def body(agg_ref, deg_ref, h_ref, ws_ref, wn_ref, b_ref, o_ref):
        agg = agg_ref[0] + agg_ref[1]
        d = deg_ref[0, :, :1] + deg_ref[1, :, :1]
        hn = agg / jnp.maximum(d, 1.0)
        acc = jnp.dot(h_ref[...], ws_ref[...], preferred_element_type=jnp.float32)
        acc += jnp.dot(hn, wn_ref[...], preferred_element_type=jnp.float32)
        acc += b_ref[...]
        o_ref[...] = jnp.maximum(acc, 0.0) if relu else acc

    return body


def _sage_mm(aggp, degp, h, Ws, Wn, b, relu):
    return pl.pallas_call(
        _make_mm_body(relu),
        grid=(N // N_BLK,),
        in_specs=[
            pl.BlockSpec((NC, N_BLK, D), lambda i: (0, i, 0)),
            pl.BlockSpec((NC, N_BLK, 8), lambda i: (0, i, 0)),
            pl.BlockSpec((N_BLK, D), lambda i: (i, 0)),
            pl.BlockSpec((D, D), lambda i: (0, 0)),
            pl.BlockSpec((D, D), lambda i: (0, 0)),
            pl.BlockSpec((1, D), lambda i: (0, 0)),
        ],
        out_specs=pl.BlockSpec((N_BLK, D), lambda i: (i, 0)),
        out_shape=jax.ShapeDtypeStruct((N, D), jnp.float32),
    )(aggp, degp, h, Ws, Wn, b.reshape(1, D))


@jax.jit
def _run(node_idx, src, dst, etype, pre_embed, rel_weght,
         W_self0, W_neigh0, b0, W_self1, W_neigh1, b1):
    i32 = jnp.int32
    nidx = jnp.concatenate(
        [node_idx.astype(i32), jnp.zeros((NODE_PAD - N,), i32)]
    ).reshape(NODE_PAD // CHG, CHG)
    src_p = jnp.concatenate(
        [src.astype(i32), jnp.zeros((E_PAD - E,), i32)]
    ).reshape(E_PAD // CH, CH)
    dst_p = jnp.concatenate(
        [dst.astype(i32), jnp.full((E_PAD - E,), DUMMY, i32)]
    ).reshape(E_PAD // CH, CH)
    typ_p = jnp.concatenate(
        [etype.astype(i32), jnp.zeros((E_PAD - E,), i32)]
    ).reshape(E_PAD // CH, CH)
    idx_p = jnp.stack([src_p, dst_p, typ_p], axis=1)

    h0p = _gather_nodes(pre_embed, nidx)
    h0 = h0p[:N]

    degp = _deg_count(idx_p)[:, :, :8]
    agg0 = _edge_agg(h0, rel_weght, idx_p)
    h1 = _sage_mm(agg0, degp, h0, W_self0, W_neigh0, b0, True)

    agg1 = _edge_agg(h1, rel_weght, idx_p)
    h2 = _sage_mm(agg1, degp, h1, W_self1, W_neigh1, b1, False)
    return h2


def kernel(node_idx, edge_index, edge_type, pre_embed, rel_weght,
           W_self0, W_neigh0, b0, W_self1, W_neigh1, b1):
    return _run(node_idx, edge_index[0], edge_index[1], edge_type,
                pre_embed, rel_weght,
                W_self0, W_neigh0, b0, W_self1, W_neigh1, b1)


# speedup vs baseline: 5.2580x; 1.0014x over previous
import functools

import jax
import jax.numpy as jnp
from jax import lax
from jax.experimental import pallas as pl
from jax.experimental.pallas import tpu as pltpu
from jax.experimental.pallas import tpu_sc as plsc

N = 10000
E = 320000
D = 128
R = 32
NC, NS = 2, 16
NW = NC * NS
CH = 80
CHG = 128

N_PAD = 10240
ROWS_PER_TILE = N_PAD // NS
NODE_PAD = NW * 3 * CHG
E_PAD = NW * 126 * CH
NCHUNK = E_PAD // (NW * CH)
DUMMY = N

_mesh = plsc.VectorSubcoreMesh(core_axis_name="c", subcore_axis_name="s")


def _zero_rows(buf, ncols):
    z = jnp.zeros((16,), jnp.float32)

    def row(i, _):
        for d in range(ncols // 16):
            buf[i, pl.ds(d * 16, 16)] = z
        return 0

    lax.fori_loop(0, buf.shape[0], row, 0)


@functools.partial(
    pl.kernel,
    out_type=jax.ShapeDtypeStruct((NODE_PAD, D), jnp.float32),
    mesh=_mesh,
    scratch_types=[
        pltpu.VMEM((CHG,), jnp.int32),
        pltpu.VMEM((CHG,), jnp.int32),
        pltpu.VMEM((CHG, D), jnp.float32),
        pltpu.VMEM((CHG, D), jnp.float32),
        pltpu.SemaphoreType.DMA,
        pltpu.SemaphoreType.DMA,
    ],
)
def _gather_nodes(table_hbm, nidx_hbm, out_hbm, i0, i1, r0, r1, s0, s1):
    wid = lax.axis_index("c") * NS + lax.axis_index("s")
    idx = (i0, i1)
    rows = (r0, r1)
    sems = (s0, s1)
    nch = NODE_PAD // (NW * CHG)
    base = wid * nch
    pltpu.sync_copy(nidx_hbm.at[base], i0)
    pltpu.async_copy(table_hbm.at[i0], r0, s0)
    for j in range(nch):
        b = j % 2
        if j + 1 < nch:
            b2 = (j + 1) % 2
            pltpu.sync_copy(nidx_hbm.at[base + j + 1], idx[b2])
            pltpu.async_copy(table_hbm.at[idx[b2]], rows[b2], sems[b2])
        pltpu.make_async_copy(table_hbm.at[idx[b]], rows[b], sems[b]).wait()
        pltpu.sync_copy(rows[b], out_hbm.at[pl.ds((base + j) * CHG, CHG)])


@functools.partial(
    pl.kernel,
    out_type=jax.ShapeDtypeStruct((NC, N_PAD, D), jnp.float32),
    mesh=_mesh,
    scratch_types=[
        pltpu.VMEM((CH,), jnp.int32),
        pltpu.VMEM((CH,), jnp.int32),
        pltpu.VMEM((CH, D), jnp.float32),
        pltpu.VMEM_SHARED((N_PAD, D), jnp.float32),
        pltpu.SemaphoreType.DMA,
        pltpu.SemaphoreType.DMA,
    ],
)
def _deg_count(idx_hbm, out_hbm, d0, d1, ones, deg_s, s0, s1):
    c = lax.axis_index("c")
    s = lax.axis_index("s")
    wid = c * NS + s
    base = s * ROWS_PER_TILE

    _zero_rows(ones, D)
    for k in range(ROWS_PER_TILE // CH):
        pltpu.sync_copy(ones, deg_s.at[pl.ds(base + k * CH, CH)])
    o = jnp.full((16,), 1.0, jnp.float32)

    def onesrow(i, _):
        ones[i, pl.ds(0, 16)] = o
        return 0

    lax.fori_loop(0, CH, onesrow, 0)
    plsc.subcore_barrier()

    didx = (d0, d1)
    sems = (s0, s1)
    row0 = wid * NCHUNK

    def pair(p, _):
        for b in range(2):
            t = 2 * p + b

            @pl.when(p >= 1)
            def _():
                pltpu.make_async_copy(ones, deg_s.at[didx[b]], sems[b]).wait()

            pltpu.sync_copy(idx_hbm.at[row0 + t, 1], didx[b])
            pltpu.async_copy(ones, deg_s.at[didx[b]], sems[b], add=True)
        return 0

    lax.fori_loop(0, NCHUNK // 2, pair, 0)
    for b in range(2):
        pltpu.make_async_copy(ones, deg_s.at[didx[b]], sems[b]).wait()
    plsc.subcore_barrier()

    for k in range(ROWS_PER_TILE // CH):
        sl = pl.ds(base + k * CH, CH)
        pltpu.sync_copy(deg_s.at[sl], out_hbm.at[c, sl])


@functools.partial(
    pl.kernel,
    out_type=jax.ShapeDtypeStruct((NC, N_PAD, D), jnp.float32),
    mesh=_mesh,
    scratch_types=[
        pltpu.VMEM((3, CH), jnp.int32),
        pltpu.VMEM((3, CH), jnp.int32),
        pltpu.VMEM((CH, D), jnp.float32),
        pltpu.VMEM((CH, D), jnp.float32),
        pltpu.VMEM((CH, D), jnp.float32),
        pltpu.VMEM((CH, D), jnp.float32),
        pltpu.VMEM_SHARED((R, D), jnp.float32),
        pltpu.VMEM_SHARED((N_PAD, D), jnp.float32),
        pltpu.SemaphoreType.DMA,
        pltpu.SemaphoreType.DMA,
        pltpu.SemaphoreType.DMA,
        pltpu.SemaphoreType.DMA,
        pltpu.SemaphoreType.DMA,
        pltpu.SemaphoreType.DMA,
    ],
)
def _edge_agg(h_hbm, rel_hbm, idx_hbm, agg_out,
              ib0, ib1, hb0, hb1, wb0, wb1, rel_s, agg_s,
              hs0, hs1, ws0, ws1, ss0, ss1):
    c = lax.axis_index("c")
    s = lax.axis_index("s")
    wid = c * NS + s
    base = s * ROWS_PER_TILE

    _zero_rows(hb0, D)
    for k in range(ROWS_PER_TILE // CH):
        pltpu.sync_copy(hb0, agg_s.at[pl.ds(base + k * CH, CH)])

    @pl.when(s == 0)
    def _():
        pltpu.sync_copy(rel_hbm, rel_s)

    plsc.subcore_barrier()

    ib = (ib0, ib1)
    hb = (hb0, hb1)
    wb = (wb0, wb1)
    hs = (hs0, hs1)
    ws = (ws0, ws1)
    ss = (ss0, ss1)
    row0 = wid * NCHUNK

    pltpu.sync_copy(idx_hbm.at[row0], ib0)
    pltpu.async_copy(h_hbm.at[ib0.at[0]], hb0, hs0)
    pltpu.async_copy(rel_s.at[ib0.at[2]], wb0, ws0)

    def pair(p, _):
        for b in range(2):
            t = 2 * p + b
            b2 = 1 - b
            pltpu.make_async_copy(h_hbm.at[ib[b].at[0]], hb[b], hs[b]).wait()
            pltpu.make_async_copy(rel_s.at[ib[b].at[2]], wb[b], ws[b]).wait()

            @pl.when(t >= 1)
            def _():
                pltpu.make_async_copy(hb[b2], agg_s.at[ib[b2].at[1]], ss[b2]).wait()

            @pl.when(t + 1 < NCHUNK)
            def _():
                pltpu.sync_copy(idx_hbm.at[row0 + t + 1], ib[b2])
                pltpu.async_copy(h_hbm.at[ib[b2].at[0]], hb[b2], hs[b2])
                pltpu.async_copy(rel_s.at[ib[b2].at[2]], wb[b2], ws[b2])

            @plsc.parallel_loop(0, CH, unroll=4)
            def _(i):
                for d in range(D // 16):
                    sl = pl.ds(d * 16, 16)
                    hb[b][i, sl] = hb[b][i, sl] * wb[b][i, sl]

            pltpu.async_copy(hb[b], agg_s.at[ib[b].at[1]], ss[b], add=True)
        return 0

    lax.fori_loop(0, NCHUNK // 2, pair, 0)
    pltpu.make_async_copy(hb1, agg_s.at[ib1.at[1]], ss1).wait()
    plsc.subcore_barrier()

    for k in range(ROWS_PER_TILE // CH):
        sl = pl.ds(base + k * CH, CH)
        pltpu.sync_copy(agg_s.at[sl], agg_out.at[c, sl])


N_BLK = 2000


def _make_mm_body(relu):
    def body(agg_ref, deg_ref, h_ref, ws_ref, wn_ref, b_ref, o_ref):
        agg = agg_ref[0] + agg_ref[1]
        d = deg_ref[0, :, :1] + deg_ref[1, :, :1]
        hn = agg / jnp.maximum(d, 1.0)
        acc = jnp.dot(h_ref[...], ws_ref[...], preferred_element_type=jnp.float32)
        acc += jnp.dot(hn, wn_ref[...], preferred_element_type=jnp.float32)
        acc += b_ref[...]
        o_ref[...] = jnp.maximum(acc, 0.0) if relu else acc

    return body


def _sage_mm(aggp, degp, h, Ws, Wn, b, relu):
    return pl.pallas_call(
        _make_mm_body(relu),
        grid=(N // N_BLK,),
        in_specs=[
            pl.BlockSpec((NC, N_BLK, D), lambda i: (0, i, 0)),
            pl.BlockSpec((NC, N_BLK, 8), lambda i: (0, i, 0)),
            pl.BlockSpec((N_BLK, D), lambda i: (i, 0)),
            pl.BlockSpec((D, D), lambda i: (0, 0)),
            pl.BlockSpec((D, D), lambda i: (0, 0)),
            pl.BlockSpec((1, D), lambda i: (0, 0)),
        ],
        out_specs=pl.BlockSpec((N_BLK, D), lambda i: (i, 0)),
        out_shape=jax.ShapeDtypeStruct((N, D), jnp.float32),
    )(aggp, degp, h, Ws, Wn, b.reshape(1, D))


@jax.jit
def _run(node_idx, src, dst, etype, pre_embed, rel_weght,
         W_self0, W_neigh0, b0, W_self1, W_neigh1, b1):
    i32 = jnp.int32
    nidx = jnp.concatenate(
        [node_idx.astype(i32), jnp.zeros((NODE_PAD - N,), i32)]
    ).reshape(NODE_PAD // CHG, CHG)
    src_p = jnp.concatenate(
        [src.astype(i32), jnp.zeros((E_PAD - E,), i32)]
    ).reshape(E_PAD // CH, CH)
    dst_p = jnp.concatenate(
        [dst.astype(i32), jnp.full((E_PAD - E,), DUMMY, i32)]
    ).reshape(E_PAD // CH, CH)
    typ_p = jnp.concatenate(
        [etype.astype(i32), jnp.zeros((E_PAD - E,), i32)]
    ).reshape(E_PAD // CH, CH)
    idx_p = jnp.stack([src_p, dst_p, typ_p], axis=1)

    h0p = _gather_nodes(pre_embed, nidx)
    h0 = h0p[:N]

    degp = _deg_count(idx_p)[:, :, :8]
    agg0 = _edge_agg(h0, rel_weght, idx_p)
    h1 = _sage_mm(agg0, degp, h0, W_self0, W_neigh0, b0, True)

    agg1 = _edge_agg(h1, rel_weght, idx_p)
    h2 = _sage_mm(agg1, degp, h1, W_self1, W_neigh1, b1, False)
    return h2


def kernel(node_idx, edge_index, edge_type, pre_embed, rel_weght,
           W_self0, W_neigh0, b0, W_self1, W_neigh1, b1):
    return _run(node_idx, edge_index[0], edge_index[1], edge_type,
                pre_embed, rel_weght,
                W_self0, W_neigh0, b0, W_self1, W_neigh1, b1)
